# baseline (speedup 1.0000x reference)
import functools

import jax
import jax.numpy as jnp
from jax.experimental import pallas as pl
from jax.experimental.pallas import tpu as pltpu

E = 8
B = 1024
BLK = 32
NBLK = 40
SPAD = NBLK * BLK
LDA = 0.1
MAX_ITER = 25
HIGHEST = jax.lax.Precision.HIGHEST


def _route_kernel(x_ref, wr_ref, br_ref,
                  sel0_ref, gate_ref, dest_ref, bexp_ref, loss_ref, mc_ref):
    x = x_ref[...]
    wr = wr_ref[...]
    sel = jax.lax.dot_general(x, wr, (((1,), (0,)), ((), ())),
                              precision=HIGHEST)
    sel = sel + 64.0 * br_ref[...]
    m = jnp.max(sel, axis=1, keepdims=True)
    ex = jnp.exp(sel - m)
    ssm = ex / jnp.sum(ex, axis=1, keepdims=True)

    mx = jnp.max(ssm)
    q = jnp.exp(ssm / (LDA * mx))
    v = jnp.ones((B, 1), jnp.float32)
    u = jnp.ones((1, E), jnp.float32)
    for _ in range(MAX_ITER):
        kv = jnp.sum(q * v, axis=0, keepdims=True)
        u = (float(B) / float(E)) / (kv + 1e-9)
        ktu = jnp.sum(q * u, axis=1, keepdims=True)
        v = 1.0 / (ktu + 1e-9)
    pi = u * q * v

    rmax = jnp.max(pi, axis=1, keepdims=True)
    keep = pi >= rmax
    lane = jax.lax.broadcasted_iota(jnp.int32, (B, E), 1)
    idx = jnp.min(jnp.where(keep, lane, E), axis=1, keepdims=True)
    onehot = (lane == idx).astype(jnp.float32)
    gate = jnp.sum(ssm * onehot, axis=1, keepdims=True)

    mc = jnp.sum(onehot, axis=0, keepdims=True)
    proxy = jnp.mean(ssm, axis=0, keepdims=True)
    loss_ref[...] = (jnp.sum(proxy * (mc / float(B)), keepdims=True)
                     / float(E) * float(E * E))

    pc = jnp.ceil(mc / float(BLK)) * float(BLK)
    ii = jax.lax.broadcasted_iota(jnp.int32, (E, E), 0)
    jj = jax.lax.broadcasted_iota(jnp.int32, (E, E), 1)
    lt = (ii < jj).astype(jnp.float32)
    po = jax.lax.dot_general(pc, lt, (((1,), (0,)), ((), ())),
                             precision=HIGHEST)

    acc = jnp.concatenate([jnp.zeros((1, E), jnp.float32), onehot[:-1, :]], 0)
    k = 1
    while k < B:
        acc = acc + jnp.concatenate(
            [jnp.zeros((k, E), jnp.float32), acc[:-k, :]], 0)
        k *= 2
    rank = jnp.sum(acc * onehot, axis=1, keepdims=True)
    po_b = jnp.sum(po * onehot, axis=1, keepdims=True)
    dest_ref[...] = (po_b + rank).astype(jnp.int32)

    po_incl = po + pc
    srow = (jax.lax.broadcasted_iota(jnp.int32, (NBLK, E), 0)
            .astype(jnp.float32) * float(BLK))
    be = jnp.sum((po_incl <= srow).astype(jnp.int32), axis=1, keepdims=True)
    bexp_ref[...] = jnp.minimum(be, E - 1)

    sel0_ref[...] = onehot
    gate_ref[...] = gate
    mc_ref[...] = mc


def _scatter_kernel(dest_sm, x_ref, xs_ref):
    del dest_sm
    xs_ref[...] = x_ref[...]


DEF = jax.lax.Precision.DEFAULT


def _dense_kernel(be_sm, xs_ref, w1_ref, b1_ref, w2_ref, b2_ref,
                  wfc_ref, bfc_ref, out_ref):
    del be_sm
    xp = xs_ref[...]
    im1 = jnp.concatenate([xp[:, 0:16, :], xp[:, 1:17, 0:96]], axis=-1)
    im1 = im1.reshape(BLK * 16, 288)
    h1 = jax.lax.dot_general(im1, w1_ref[0], (((1,), (0,)), ((), ())),
                             precision=DEF)
    h1 = jax.nn.relu(h1 + b1_ref[0])
    h1 = h1.reshape(BLK, 16, 16, 64)
    hp = jnp.pad(h1, ((0, 0), (0, 2), (0, 2), (0, 0)))
    y2 = hp.reshape(BLK, 9, 2, 9, 2, 64)
    taps2 = [y2[:, kh // 2:kh // 2 + 8, kh % 2,
                kw // 2:kw // 2 + 8, kw % 2, :]
             for kh in range(3) for kw in range(3)]
    im2 = jnp.concatenate(taps2, axis=-1).reshape(BLK * 64, 576)
    h2 = jax.lax.dot_general(im2, w2_ref[0], (((1,), (0,)), ((), ())),
                             precision=DEF)
    h2 = jax.nn.relu(h2 + b2_ref[0])
    pooled = jnp.mean(h2.reshape(BLK, 64, 64), axis=1)
    out = jax.lax.dot_general(pooled, wfc_ref[0], (((1,), (0,)), ((), ())),
                              precision=DEF)
    out_ref[...] = out + bfc_ref[0]


def _gather_kernel(dest_sm, gate_sm, os_ref, out_ref):
    del dest_sm
    g = jax.lax.bitcast_convert_type(gate_sm[pl.program_id(0)], jnp.float32)
    out_ref[...] = os_ref[...] * g


def kernel(x, W_router, b_router, Wc1, bc1, Wc2, bc2, Wfc, bfc):
    f32 = jnp.float32
    x_cl = x.transpose(0, 2, 3, 1).reshape(B, 3072)
    wr_full = jnp.tile(W_router, (1, 1, 8, 8))
    wr_cl = wr_full.transpose(2, 3, 1, 0).reshape(3072, E)
    br = b_router.reshape(1, E).astype(f32)

    sel0, gate, dest, bexp, loss, mc = pl.pallas_call(
        _route_kernel,
        out_shape=(
            jax.ShapeDtypeStruct((B, E), f32),
            jax.ShapeDtypeStruct((B, 1), f32),
            jax.ShapeDtypeStruct((B, 1), jnp.int32),
            jax.ShapeDtypeStruct((NBLK, 1), jnp.int32),
            jax.ShapeDtypeStruct((1, 1), f32),
            jax.ShapeDtypeStruct((1, E), f32),
        ),
    )(x_cl, wr_cl, br)

    dest1 = dest.reshape(B)
    x_rp = jnp.pad(x_cl.reshape(B, 16, 192), ((0, 0), (0, 1), (0, 0)))
    xs = pl.pallas_call(
        _scatter_kernel,
        grid_spec=pltpu.PrefetchScalarGridSpec(
            num_scalar_prefetch=1,
            grid=(B,),
            in_specs=[pl.BlockSpec((1, 17, 192), lambda b, d: (b, 0, 0))],
            out_specs=pl.BlockSpec((1, 17, 192), lambda b, d: (d[b], 0, 0)),
        ),
        out_shape=jax.ShapeDtypeStruct((SPAD, 17, 192), f32),
    )(dest1, x_rp)

    w1v = Wc1.transpose(0, 3, 4, 2, 1)
    kwi = jnp.arange(3)[:, None, None]
    wi = jnp.arange(32)[None, :, None]
    owi = jnp.arange(16)[None, None, :]
    msk = (wi == 2 * owi + kwi).astype(f32)
    t1 = jnp.einsum('ehkcd,kwo->ehwcod', w1v, msk).reshape(E, 288, 1024)
    b1t = jnp.tile(bc1, (1, 16)).reshape(E, 1, 1024)
    w2 = Wc2.transpose(0, 3, 4, 2, 1).reshape(E, 576, 64)
    b2 = bc2.reshape(E, 1, 64)
    bf = bfc.reshape(E, 1, 1000)
    be1 = bexp.reshape(NBLK)

    def _wmap(s, be):
        return (be[s], 0, 0)

    os_ = pl.pallas_call(
        _dense_kernel,
        grid_spec=pltpu.PrefetchScalarGridSpec(
            num_scalar_prefetch=1,
            grid=(NBLK,),
            in_specs=[
                pl.BlockSpec((BLK, 17, 192), lambda s, be: (s, 0, 0)),
                pl.BlockSpec((1, 288, 1024), _wmap),
                pl.BlockSpec((1, 1, 1024), _wmap),
                pl.BlockSpec((1, 576, 64), _wmap),
                pl.BlockSpec((1, 1, 64), _wmap),
                pl.BlockSpec((1, 64, 1000), _wmap),
                pl.BlockSpec((1, 1, 1000), _wmap),
            ],
            out_specs=pl.BlockSpec((BLK, 1000), lambda s, be: (s, 0)),
        ),
        out_shape=jax.ShapeDtypeStruct((SPAD, 1000), f32),
    )(be1, xs, t1, b1t, w2, b2, Wfc, bf)

    gate_i = jax.lax.bitcast_convert_type(gate.reshape(B), jnp.int32)
    output = pl.pallas_call(
        _gather_kernel,
        grid_spec=pltpu.PrefetchScalarGridSpec(
            num_scalar_prefetch=2,
            grid=(B,),
            in_specs=[pl.BlockSpec((1, 1, 1000), lambda b, d, g: (d[b], 0, 0))],
            out_specs=pl.BlockSpec((1, 1, 1000), lambda b, d, g: (b, 0, 0)),
        ),
        out_shape=jax.ShapeDtypeStruct((B, 1, 1000), f32),
    )(dest1, gate_i, os_.reshape(SPAD, 1, 1000)).reshape(B, 1000)

    return (output, sel0, loss.reshape(()), mc)


# speedup vs baseline: 3.1872x; 3.1872x over previous
import functools

import jax
import jax.numpy as jnp
from jax import lax
from jax.experimental import pallas as pl
from jax.experimental.pallas import tpu as pltpu
from jax.experimental.pallas import tpu_sc as plsc

E = 8
B = 1024
BLK = 32
NBLK = 40
SPAD = NBLK * BLK
LDA = 0.1
MAX_ITER = 25
HIGHEST = jax.lax.Precision.HIGHEST


def _route_kernel(x_ref, wr_ref, br_ref,
                  sel0_ref, dest_ref, inv_ref, gs_ref, bexp_ref,
                  loss_ref, mc_ref):
    x = x_ref[...]
    wr = wr_ref[...]
    sel = jax.lax.dot_general(x, wr, (((1,), (0,)), ((), ())),
                              precision=jax.lax.Precision.DEFAULT)
    sel = sel + 64.0 * br_ref[...]
    m = jnp.max(sel, axis=1, keepdims=True)
    ex = jnp.exp(sel - m)
    ssm = ex / jnp.sum(ex, axis=1, keepdims=True)

    mx = jnp.max(ssm)
    q = jnp.exp((ssm / mx) / jnp.float32(LDA))
    v = jnp.ones((B, 1), jnp.float32)
    u = jnp.ones((1, E), jnp.float32)
    for _ in range(MAX_ITER):
        kv = jnp.sum(q * v, axis=0, keepdims=True)
        u = (float(B) / float(E)) / (kv + 1e-9)
        ktu = jnp.sum(q * u, axis=1, keepdims=True)
        v = 1.0 / (ktu + 1e-9)
    pi = u * q * v

    rmax = jnp.max(pi, axis=1, keepdims=True)
    keep = pi >= rmax
    lane = jax.lax.broadcasted_iota(jnp.int32, (B, E), 1)
    idx = jnp.min(jnp.where(keep, lane, E), axis=1, keepdims=True)
    onehot = (lane == idx).astype(jnp.float32)
    gate = jnp.sum(ssm * onehot, axis=1, keepdims=True)

    mc = jnp.sum(onehot, axis=0, keepdims=True)
    proxy = jnp.mean(ssm, axis=0, keepdims=True)
    loss_ref[...] = (jnp.sum(proxy * (mc / float(B)), keepdims=True)
                     / float(E) * float(E * E))

    pc = jnp.ceil(mc / float(BLK)) * float(BLK)
    ii = jax.lax.broadcasted_iota(jnp.int32, (E, E), 0)
    jj = jax.lax.broadcasted_iota(jnp.int32, (E, E), 1)
    lt = (ii < jj).astype(jnp.float32)
    po = jax.lax.dot_general(pc, lt, (((1,), (0,)), ((), ())),
                             precision=HIGHEST)

    acc = jnp.concatenate([jnp.zeros((1, E), jnp.float32), onehot[:-1, :]], 0)
    k = 1
    while k < B:
        acc = acc + jnp.concatenate(
            [jnp.zeros((k, E), jnp.float32), acc[:-k, :]], 0)
        k *= 2
    rank = jnp.sum(acc * onehot, axis=1, keepdims=True)
    po_b = jnp.sum(po * onehot, axis=1, keepdims=True)
    desti = (po_b + rank).astype(jnp.int32)
    dest_ref[...] = desti

    drow = jnp.transpose(desti)
    siota = jax.lax.broadcasted_iota(jnp.int32, (SPAD, B), 0)
    cmp = siota == drow
    biota = jax.lax.broadcasted_iota(jnp.int32, (SPAD, B), 1)
    inv_ref[...] = jnp.sum(jnp.where(cmp, biota, 0), axis=1, keepdims=True)
    gs_ref[...] = jnp.sum(jnp.where(cmp, jnp.transpose(gate), 0.0),
                          axis=1, keepdims=True)

    po_incl = po + pc
    srow = (jax.lax.broadcasted_iota(jnp.int32, (NBLK, E), 0)
            .astype(jnp.float32) * float(BLK))
    be = jnp.sum((po_incl <= srow).astype(jnp.int32), axis=1, keepdims=True)
    bexp_ref[...] = jnp.minimum(be, E - 1)

    sel0_ref[...] = onehot
    mc_ref[...] = mc


DEF = jax.lax.Precision.DEFAULT


def _sc_row_gather(n_rows, row_shape, chunk):
    info = plsc.get_sparse_core_info()
    nc, ns = info.num_cores, info.num_subcores
    nw = nc * ns
    per_w = n_rows // nw
    assert per_w % chunk == 0 and chunk % 8 == 0
    mesh = plsc.VectorSubcoreMesh(core_axis_name="c", subcore_axis_name="s")

    def body(table_hbm, idx_hbm, out_hbm, idx_v, rows_v, sem):
        wid = lax.axis_index("s") * nc + lax.axis_index("c")
        base = wid * per_w
        for k in range(per_w // chunk):
            off = base + k * chunk
            pltpu.sync_copy(idx_hbm.at[pl.ds(off, chunk)], idx_v)
            pltpu.async_copy(table_hbm.at[idx_v], rows_v, sem).wait()
            pltpu.sync_copy(rows_v, out_hbm.at[pl.ds(off, chunk)])

    return pl.kernel(
        body, mesh=mesh,
        out_type=jax.ShapeDtypeStruct((n_rows,) + row_shape, jnp.float32),
        scratch_types=[
            pltpu.VMEM((chunk,), jnp.int32),
            pltpu.VMEM((chunk,) + row_shape, jnp.float32),
            pltpu.SemaphoreType.DMA,
        ],
    )


def _dense_kernel(be_sm, xs_ref, gs_ref, w1_ref, b1_ref, w2_ref, b2_ref,
                  wfc_ref, bfc_ref, out_ref):
    del be_sm
    xf = xs_ref[...]
    im1 = jnp.stack([xf[:, o * 192:o * 192 + 288] for o in range(16)],
                    axis=1).reshape(BLK * 16, 288)
    h1 = jax.lax.dot_general(im1, w1_ref[0], (((1,), (0,)), ((), ())),
                             precision=DEF)
    h1 = jax.nn.relu(h1 + b1_ref[0])
    h1 = h1.reshape(BLK, 16, 16, 64)
    hp = jnp.pad(h1, ((0, 0), (0, 2), (0, 2), (0, 0)))
    y2 = hp.reshape(BLK, 9, 2, 9, 2, 64)
    taps2 = [y2[:, kh // 2:kh // 2 + 8, kh % 2,
                kw // 2:kw // 2 + 8, kw % 2, :]
             for kh in range(3) for kw in range(3)]
    im2 = jnp.concatenate(taps2, axis=-1).reshape(BLK * 64, 576)
    h2 = jax.lax.dot_general(im2, w2_ref[0], (((1,), (0,)), ((), ())),
                             precision=DEF)
    h2 = jax.nn.relu(h2 + b2_ref[0])
    pooled = jnp.mean(h2.reshape(BLK, 64, 64), axis=1)
    out = jax.lax.dot_general(pooled, wfc_ref[0], (((1,), (0,)), ((), ())),
                              precision=DEF)
    out = (out + bfc_ref[0]) * gs_ref[...]
    out_ref[...] = jnp.pad(out, ((0, 0), (0, 24)))


def kernel(x, W_router, b_router, Wc1, bc1, Wc2, bc2, Wfc, bfc):
    f32 = jnp.float32
    x_cl = x.transpose(0, 2, 3, 1).reshape(B, 3072)
    wr_full = jnp.tile(W_router, (1, 1, 8, 8))
    wr_cl = wr_full.transpose(2, 3, 1, 0).reshape(3072, E)
    br = b_router.reshape(1, E).astype(f32)

    sel0, dest, inv, gs, bexp, loss, mc = pl.pallas_call(
        _route_kernel,
        out_shape=(
            jax.ShapeDtypeStruct((B, E), f32),
            jax.ShapeDtypeStruct((B, 1), jnp.int32),
            jax.ShapeDtypeStruct((SPAD, 1), jnp.int32),
            jax.ShapeDtypeStruct((SPAD, 1), f32),
            jax.ShapeDtypeStruct((NBLK, 1), jnp.int32),
            jax.ShapeDtypeStruct((1, 1), f32),
            jax.ShapeDtypeStruct((1, E), f32),
        ),
    )(x_cl, wr_cl, br)

    x_sc = jnp.pad(x_cl, ((0, 0), (0, 256)))
    xs = _sc_row_gather(SPAD, (3328,), 8)(x_sc, inv.reshape(SPAD))

    w1v = Wc1.transpose(0, 3, 4, 2, 1)
    kwi = jnp.arange(3)[:, None, None]
    wi = jnp.arange(32)[None, :, None]
    owi = jnp.arange(16)[None, None, :]
    msk = (wi == 2 * owi + kwi).astype(f32)
    t1 = jnp.einsum('ehkcd,kwo->ehwcod', w1v, msk).reshape(E, 288, 1024)
    b1t = jnp.tile(bc1, (1, 16)).reshape(E, 1, 1024)
    w2 = Wc2.transpose(0, 3, 4, 2, 1).reshape(E, 576, 64)
    b2 = bc2.reshape(E, 1, 64)
    bf = bfc.reshape(E, 1, 1000)
    be1 = bexp.reshape(NBLK)

    def _wmap(s, be):
        return (be[s], 0, 0)

    os_ = pl.pallas_call(
        _dense_kernel,
        grid_spec=pltpu.PrefetchScalarGridSpec(
            num_scalar_prefetch=1,
            grid=(NBLK,),
            in_specs=[
                pl.BlockSpec((BLK, 3328), lambda s, be: (s, 0)),
                pl.BlockSpec((BLK, 1), lambda s, be: (s, 0)),
                pl.BlockSpec((1, 288, 1024), _wmap),
                pl.BlockSpec((1, 1, 1024), _wmap),
                pl.BlockSpec((1, 576, 64), _wmap),
                pl.BlockSpec((1, 1, 64), _wmap),
                pl.BlockSpec((1, 64, 1000), _wmap),
                pl.BlockSpec((1, 1, 1000), _wmap),
            ],
            out_specs=pl.BlockSpec((BLK, 1024), lambda s, be: (s, 0)),
        ),
        out_shape=jax.ShapeDtypeStruct((SPAD, 1024), f32),
    )(be1, xs, gs, t1, b1t, w2, b2, Wfc, bf)

    output = _sc_row_gather(B, (1024,), 32)(os_, dest.reshape(B))[:, :1000]

    return (output, sel0, loss.reshape(()), mc)
